# Initial kernel scaffold; baseline (speedup 1.0000x reference)
import jax
import jax.numpy as jnp
from jax.experimental import pallas as pl


def kernel(block_outputs, pos_edge_index, neg_edge_index):
    raise NotImplementedError("write your pallas kernel here")


# speedup vs baseline: 5.7238x; 5.7238x over previous
import functools

import jax
import jax.numpy as jnp
from jax import lax
from jax.experimental import pallas as pl
from jax.experimental.pallas import tpu as pltpu
from jax.experimental.pallas import tpu_sc as plsc

_LANES = 16


def _pick_chunk(per_worker: int) -> int:
    for c in range(128, 0, -8):
        if per_worker % c == 0:
            return c
    raise ValueError(f"no valid chunk for per_worker={per_worker}")


@functools.lru_cache(maxsize=None)
def _make_sc_loss(n_nodes: int, d_feat: int, e_pos: int, e_neg: int):
    info = plsc.get_sparse_core_info()
    nc, ns = info.num_cores, info.num_subcores
    nw = nc * ns
    assert d_feat % _LANES == 0
    assert e_pos % nw == 0 and e_neg % nw == 0
    per_p = e_pos // nw
    per_n = e_neg // nw
    chunk_p = _pick_chunk(per_p)
    chunk_n = _pick_chunk(per_n)
    chunk = min(chunk_p, chunk_n)
    n_feat_vecs = d_feat // _LANES
    inv_total = 1.0 / float(e_pos + e_neg)

    mesh = plsc.VectorSubcoreMesh(core_axis_name="c", subcore_axis_name="s")

    @functools.partial(
        pl.kernel,
        mesh=mesh,
        out_type=jax.ShapeDtypeStruct((nw, _LANES), jnp.float32),
        scratch_types=[
            pltpu.VMEM((max(per_p, per_n),), jnp.int32),
            pltpu.VMEM((max(per_p, per_n),), jnp.int32),
            pltpu.VMEM((chunk, d_feat), jnp.float32),
            pltpu.VMEM((chunk, d_feat), jnp.float32),
            pltpu.VMEM((_LANES,), jnp.float32),
            pltpu.SemaphoreType.DMA,
            pltpu.SemaphoreType.DMA,
        ],
    )
    def sc_loss(h_hbm, ps_hbm, pd_hbm, ns_hbm, nd_hbm, out_hbm,
                idx_s, idx_d, rows_s, rows_d, outv, sem_s, sem_d):
        wid = lax.axis_index("s") * nc + lax.axis_index("c")

        def accumulate(src_hbm, dst_hbm, per_w, acc):
            base = wid * per_w
            pltpu.sync_copy(src_hbm.at[pl.ds(base, per_w)], idx_s.at[pl.ds(0, per_w)])
            pltpu.sync_copy(dst_hbm.at[pl.ds(base, per_w)], idx_d.at[pl.ds(0, per_w)])

            def chunk_body(ci, a):
                off = ci * chunk
                cp_s = pltpu.async_copy(
                    h_hbm.at[idx_s.at[pl.ds(off, chunk)]], rows_s, sem_s)
                cp_d = pltpu.async_copy(
                    h_hbm.at[idx_d.at[pl.ds(off, chunk)]], rows_d, sem_d)
                cp_s.wait()
                cp_d.wait()

                def edge_body(ei, aa):
                    for dd in range(n_feat_vecs):
                        aa = aa + (rows_s[ei, pl.ds(dd * _LANES, _LANES)]
                                   * rows_d[ei, pl.ds(dd * _LANES, _LANES)])
                    return aa

                return lax.fori_loop(0, chunk, edge_body, a)

            return lax.fori_loop(0, per_w // chunk, chunk_body, acc)

        zero = jnp.zeros((_LANES,), jnp.float32)
        acc_p = accumulate(ps_hbm, pd_hbm, per_p, zero)
        acc_n = accumulate(ns_hbm, nd_hbm, per_n, zero)
        sum_label_score = 1.0 * acc_p + 0.0 * acc_n
        sum_label_lse = 1.0 * acc_p + 0.0 * acc_n
        outv[...] = (sum_label_score - sum_label_lse) * (-inv_total)
        pltpu.sync_copy(outv, out_hbm.at[wid])

    return sc_loss


def kernel(block_outputs, pos_edge_index, neg_edge_index):
    h = block_outputs
    ps = pos_edge_index[0].astype(jnp.int32)
    pd = pos_edge_index[1].astype(jnp.int32)
    ns = neg_edge_index[0].astype(jnp.int32)
    nd = neg_edge_index[1].astype(jnp.int32)
    f = _make_sc_loss(h.shape[0], h.shape[1], ps.shape[0], ns.shape[0])
    partials = f(h, ps, pd, ns, nd)
    return jnp.sum(partials)


# speedup vs baseline: 8.1877x; 1.4305x over previous
import functools

import jax
import jax.numpy as jnp
from jax import lax
from jax.experimental import pallas as pl
from jax.experimental.pallas import tpu as pltpu
from jax.experimental.pallas import tpu_sc as plsc

_LANES = 16


def _pick_chunk(per_worker: int) -> int:
    for c in range(128, 0, -8):
        if per_worker % c == 0:
            return c
    raise ValueError(f"no valid chunk for per_worker={per_worker}")


@functools.lru_cache(maxsize=None)
def _make_sc_loss(n_nodes: int, d_feat: int, e_pos: int, e_neg: int):
    info = plsc.get_sparse_core_info()
    nc, ns = info.num_cores, info.num_subcores
    nw = nc * ns
    assert d_feat % _LANES == 0
    assert e_pos % nw == 0 and e_neg % nw == 0
    per_p = e_pos // nw
    per_n = e_neg // nw
    chunk = min(_pick_chunk(per_p), _pick_chunk(per_n))
    n_feat_vecs = d_feat // _LANES
    inv_total = 1.0 / float(e_pos + e_neg)

    mesh = plsc.VectorSubcoreMesh(core_axis_name="c", subcore_axis_name="s")

    @functools.partial(
        pl.kernel,
        mesh=mesh,
        out_type=jax.ShapeDtypeStruct((nw, _LANES), jnp.float32),
        scratch_types=[
            pltpu.VMEM((max(per_p, per_n),), jnp.int32),
            pltpu.VMEM((max(per_p, per_n),), jnp.int32),
            pltpu.VMEM((chunk, d_feat), jnp.float32),
            pltpu.VMEM((chunk, d_feat), jnp.float32),
            pltpu.VMEM((chunk, d_feat), jnp.float32),
            pltpu.VMEM((chunk, d_feat), jnp.float32),
            pltpu.VMEM((_LANES,), jnp.float32),
            pltpu.SemaphoreType.DMA,
            pltpu.SemaphoreType.DMA,
            pltpu.SemaphoreType.DMA,
            pltpu.SemaphoreType.DMA,
        ],
    )
    def sc_loss(h_hbm, ps_hbm, pd_hbm, ns_hbm, nd_hbm, out_hbm,
                idx_s, idx_d, rows_s0, rows_d0, rows_s1, rows_d1, outv,
                sem_s0, sem_d0, sem_s1, sem_d1):
        wid = lax.axis_index("s") * nc + lax.axis_index("c")
        bufs = ((rows_s0, rows_d0, sem_s0, sem_d0),
                (rows_s1, rows_d1, sem_s1, sem_d1))

        def issue(ci, b):
            rs, rd, ss, sd = bufs[b]
            off = ci * chunk
            pltpu.async_copy(h_hbm.at[idx_s.at[pl.ds(off, chunk)]], rs, ss)
            pltpu.async_copy(h_hbm.at[idx_d.at[pl.ds(off, chunk)]], rd, sd)

        def drain(b):
            rs, rd, ss, sd = bufs[b]
            pltpu.make_async_copy(h_hbm.at[idx_s.at[pl.ds(0, chunk)]], rs, ss).wait()
            pltpu.make_async_copy(h_hbm.at[idx_d.at[pl.ds(0, chunk)]], rd, sd).wait()

        def compute(b, accs):
            rs, rd = bufs[b][0], bufs[b][1]

            def edge_body(ei, a):
                return tuple(
                    a[dd] + (rs[ei, pl.ds(dd * _LANES, _LANES)]
                             * rd[ei, pl.ds(dd * _LANES, _LANES)])
                    for dd in range(n_feat_vecs))

            return lax.fori_loop(0, chunk, edge_body, accs)

        def phase(src_hbm, dst_hbm, per_w, accs):
            base = wid * per_w
            n = per_w // chunk
            pltpu.sync_copy(src_hbm.at[pl.ds(base, per_w)],
                            idx_s.at[pl.ds(0, per_w)])
            pltpu.sync_copy(dst_hbm.at[pl.ds(base, per_w)],
                            idx_d.at[pl.ds(0, per_w)])
            issue(0, 0)

            def pair_body(k, a):
                c0 = 2 * k
                drain(0)

                @pl.when(c0 + 1 < n)
                def _():
                    issue(c0 + 1, 1)

                a = compute(0, a)
                drain(1)

                @pl.when(c0 + 2 < n)
                def _():
                    issue(c0 + 2, 0)

                return compute(1, a)

            accs = lax.fori_loop(0, n // 2, pair_body, accs)
            if n % 2:
                drain(0)
                accs = compute(0, accs)
            return accs

        zero = tuple(jnp.zeros((_LANES,), jnp.float32)
                     for _ in range(n_feat_vecs))
        accs_p = phase(ps_hbm, pd_hbm, per_p, zero)
        accs_n = phase(ns_hbm, nd_hbm, per_n, zero)
        acc_p = functools.reduce(lambda x, y: x + y, accs_p)
        acc_n = functools.reduce(lambda x, y: x + y, accs_n)
        sum_label_score = 1.0 * acc_p + 0.0 * acc_n
        sum_label_lse = 1.0 * acc_p + 0.0 * acc_n
        outv[...] = (sum_label_score - sum_label_lse) * (-inv_total)
        pltpu.sync_copy(outv, out_hbm.at[wid])

    return sc_loss


def kernel(block_outputs, pos_edge_index, neg_edge_index):
    h = block_outputs
    ps = pos_edge_index[0].astype(jnp.int32)
    pd = pos_edge_index[1].astype(jnp.int32)
    ns = neg_edge_index[0].astype(jnp.int32)
    nd = neg_edge_index[1].astype(jnp.int32)
    f = _make_sc_loss(h.shape[0], h.shape[1], ps.shape[0], ns.shape[0])
    partials = f(h, ps, pd, ns, nd)
    return jnp.sum(partials)
